# Initial kernel scaffold; baseline (speedup 1.0000x reference)
import jax
import jax.numpy as jnp
from jax.experimental import pallas as pl


def kernel(x, token_table, pos_table):
    raise NotImplementedError("write your pallas kernel here")


# speedup vs baseline: 4.1256x; 4.1256x over previous
import functools

import jax
import jax.numpy as jnp
from jax import lax
from jax.experimental import pallas as pl
from jax.experimental.pallas import tpu as pltpu
from jax.experimental.pallas import tpu_sc as plsc

VOCAB = 100000
MAXLEN = 200
DIM = 32
BATCH = 4096

NUM_CORES = 2
NUM_SUBCORES = 16
NUM_WORKERS = NUM_CORES * NUM_SUBCORES

TOTAL = BATCH * MAXLEN
PER_WORKER = TOTAL // NUM_WORKERS
CHUNK = 4 * MAXLEN
NCHUNK = PER_WORKER // CHUNK


def _sc_embed(token_table, idx_flat, pos_tiled):
    mesh = plsc.VectorSubcoreMesh(core_axis_name="c", subcore_axis_name="s")

    @functools.partial(
        pl.kernel,
        mesh=mesh,
        compiler_params=pltpu.CompilerParams(use_tc_tiling_on_sc=False),
        out_type=jax.ShapeDtypeStruct((TOTAL, DIM), jnp.float32),
        scratch_types=[
            pltpu.VMEM((CHUNK,), jnp.int32),
            pltpu.VMEM((CHUNK, DIM), jnp.float32),
            pltpu.VMEM((CHUNK, DIM), jnp.float32),
            pltpu.SemaphoreType.DMA,
        ],
    )
    def k(table_hbm, idx_hbm, pos_hbm, out_hbm, idx_v, rows_v, pos_v, sem):
        wid = lax.axis_index("s") * NUM_CORES + lax.axis_index("c")
        base = wid * PER_WORKER
        pltpu.sync_copy(pos_hbm, pos_v)

        @pl.loop(0, NCHUNK)
        def _(ci):
            off = base + ci * CHUNK
            pltpu.sync_copy(idx_hbm.at[pl.ds(off, CHUNK)], idx_v)
            pltpu.async_copy(table_hbm.at[idx_v], rows_v, sem).wait()

            @pl.loop(0, CHUNK)
            def _(r):
                for c in (0, DIM // 2):
                    slc = (pl.ds(r, 1), pl.ds(c, 16))
                    rows_v.at[*slc][...] = (
                        rows_v.at[*slc][...] + pos_v.at[*slc][...]
                    )

            pltpu.sync_copy(rows_v, out_hbm.at[pl.ds(off, CHUNK)])

    return k(token_table, idx_flat, pos_tiled)


def kernel(x, token_table, pos_table):
    idx_flat = x.reshape(TOTAL).astype(jnp.int32)
    pos_tiled = jnp.tile(pos_table.astype(jnp.float32), (CHUNK // MAXLEN, 1))
    out = _sc_embed(token_table.astype(jnp.float32), idx_flat, pos_tiled)
    return out.reshape(BATCH, MAXLEN, DIM)


# speedup vs baseline: 4.5047x; 1.0919x over previous
import functools

import jax
import jax.numpy as jnp
from jax import lax
from jax.experimental import pallas as pl
from jax.experimental.pallas import tpu as pltpu
from jax.experimental.pallas import tpu_sc as plsc

VOCAB = 100000
MAXLEN = 200
DIM = 32
BATCH = 4096

NUM_CORES = 2
NUM_SUBCORES = 16
NUM_WORKERS = NUM_CORES * NUM_SUBCORES

TOTAL = BATCH * MAXLEN
PER_WORKER = TOTAL // NUM_WORKERS
CHUNK = 4 * MAXLEN
NCHUNK = PER_WORKER // CHUNK


def _sc_embed(token_table, idx_flat, pos_tiled):
    mesh = plsc.VectorSubcoreMesh(core_axis_name="c", subcore_axis_name="s")

    @functools.partial(
        pl.kernel,
        mesh=mesh,
        compiler_params=pltpu.CompilerParams(use_tc_tiling_on_sc=False),
        out_type=jax.ShapeDtypeStruct((TOTAL // 4, 4 * DIM), jnp.float32),
        scratch_types=[
            pltpu.VMEM((CHUNK,), jnp.int32),
            pltpu.VMEM((CHUNK, DIM), jnp.float32),
            pltpu.VMEM((CHUNK // 4, 4 * DIM), jnp.float32),
            pltpu.VMEM((CHUNK // 4, 4 * DIM), jnp.float32),
            pltpu.SemaphoreType.DMA,
        ],
    )
    def k(table_hbm, idx_hbm, pos_hbm, out_hbm, idx_v, rows_v, rows_out,
          pos_v, sem):
        wid = lax.axis_index("s") * NUM_CORES + lax.axis_index("c")
        base = wid * PER_WORKER
        pltpu.sync_copy(pos_hbm, pos_v)

        @pl.loop(0, NCHUNK)
        def _(ci):
            off = base + ci * CHUNK
            pltpu.sync_copy(idx_hbm.at[pl.ds(off, CHUNK)], idx_v)
            pltpu.async_copy(table_hbm.at[idx_v], rows_v, sem).wait()

            @pl.loop(0, CHUNK // 4)
            def _(ro):
                for q in range(4):
                    for c in (0, DIM // 2):
                        dst = (pl.ds(ro, 1), pl.ds(q * DIM + c, 16))
                        src = (pl.ds(4 * ro + q, 1), pl.ds(c, 16))
                        rows_out.at[*dst][...] = (
                            rows_v.at[*src][...] + pos_v.at[*dst][...]
                        )

            pltpu.sync_copy(
                rows_out, out_hbm.at[pl.ds(off // 4, CHUNK // 4)]
            )

    return k(token_table, idx_flat, pos_tiled)


def kernel(x, token_table, pos_table):
    idx_flat = x.reshape(TOTAL).astype(jnp.int32)
    pos_tiled = jnp.tile(
        pos_table.astype(jnp.float32), (CHUNK // MAXLEN, 1)
    ).reshape(CHUNK // 4, 4 * DIM)
    out = _sc_embed(token_table.astype(jnp.float32), idx_flat, pos_tiled)
    return out.reshape(BATCH, MAXLEN, DIM)


# speedup vs baseline: 4.9000x; 1.0878x over previous
import functools

import jax
import jax.numpy as jnp
from jax import lax
from jax.experimental import pallas as pl
from jax.experimental.pallas import tpu as pltpu
from jax.experimental.pallas import tpu_sc as plsc

VOCAB = 100000
MAXLEN = 200
DIM = 32
BATCH = 4096

NUM_CORES = 2
NUM_SUBCORES = 16
NUM_WORKERS = NUM_CORES * NUM_SUBCORES

TOTAL = BATCH * MAXLEN
PER_WORKER = TOTAL // NUM_WORKERS
CHUNK = 4 * MAXLEN
NCHUNK = PER_WORKER // CHUNK


def _sc_embed(token_table, idx_flat, pos_tiled):
    mesh = plsc.VectorSubcoreMesh(core_axis_name="c", subcore_axis_name="s")

    @functools.partial(
        pl.kernel,
        mesh=mesh,
        compiler_params=pltpu.CompilerParams(use_tc_tiling_on_sc=False),
        out_type=jax.ShapeDtypeStruct((TOTAL // 4, 4 * DIM), jnp.float32),
        scratch_types=[
            pltpu.VMEM((CHUNK,), jnp.int32),
            pltpu.VMEM((CHUNK,), jnp.int32),
            pltpu.VMEM((CHUNK, DIM), jnp.float32),
            pltpu.VMEM((CHUNK, DIM), jnp.float32),
            pltpu.VMEM((CHUNK // 4, 4 * DIM), jnp.float32),
            pltpu.VMEM((CHUNK // 4, 4 * DIM), jnp.float32),
            pltpu.SemaphoreType.DMA,
            pltpu.SemaphoreType.DMA,
        ],
    )
    def k(table_hbm, idx_hbm, pos_hbm, out_hbm,
          idx_a, idx_b, rows_a, rows_b, rows_out, pos_v, sem_a, sem_b):
        wid = lax.axis_index("s") * NUM_CORES + lax.axis_index("c")
        base = wid * PER_WORKER
        pltpu.sync_copy(pos_hbm, pos_v)

        def start_gather(ci, idx_v, rows_v, sem):
            off = base + ci * CHUNK
            pltpu.sync_copy(idx_hbm.at[pl.ds(off, CHUNK)], idx_v)
            pltpu.async_copy(table_hbm.at[idx_v], rows_v, sem)

        def wait_gather(idx_v, rows_v, sem):
            pltpu.make_async_copy(table_hbm.at[idx_v], rows_v, sem).wait()

        def consume(ci, rows_v):
            @pl.loop(0, CHUNK // 4)
            def _(ro):
                for q in range(4):
                    for c in (0, DIM // 2):
                        dst = (pl.ds(ro, 1), pl.ds(q * DIM + c, 16))
                        src = (pl.ds(4 * ro + q, 1), pl.ds(c, 16))
                        rows_out.at[*dst][...] = (
                            rows_v.at[*src][...] + pos_v.at[*dst][...]
                        )

            off = base + ci * CHUNK
            pltpu.sync_copy(
                rows_out, out_hbm.at[pl.ds(off // 4, CHUNK // 4)]
            )

        start_gather(0, idx_a, rows_a, sem_a)

        @pl.loop(0, NCHUNK, step=2)
        def _(c0):
            start_gather(c0 + 1, idx_b, rows_b, sem_b)
            wait_gather(idx_a, rows_a, sem_a)
            consume(c0, rows_a)

            @pl.when(c0 + 2 < NCHUNK)
            def _():
                start_gather(c0 + 2, idx_a, rows_a, sem_a)

            wait_gather(idx_b, rows_b, sem_b)
            consume(c0 + 1, rows_b)

    return k(token_table, idx_flat, pos_tiled)


def kernel(x, token_table, pos_table):
    idx_flat = x.reshape(TOTAL).astype(jnp.int32)
    pos_tiled = jnp.tile(
        pos_table.astype(jnp.float32), (CHUNK // MAXLEN, 1)
    ).reshape(CHUNK // 4, 4 * DIM)
    out = _sc_embed(token_table.astype(jnp.float32), idx_flat, pos_tiled)
    return out.reshape(BATCH, MAXLEN, DIM)


# speedup vs baseline: 4.9214x; 1.0044x over previous
import functools

import jax
import jax.numpy as jnp
from jax import lax
from jax.experimental import pallas as pl
from jax.experimental.pallas import tpu as pltpu
from jax.experimental.pallas import tpu_sc as plsc

VOCAB = 100000
MAXLEN = 200
DIM = 32
BATCH = 4096

NUM_CORES = 2
NUM_SUBCORES = 16
NUM_WORKERS = NUM_CORES * NUM_SUBCORES

TOTAL = BATCH * MAXLEN
PER_WORKER = TOTAL // NUM_WORKERS
CHUNK = 4 * MAXLEN
NCHUNK = PER_WORKER // CHUNK


def _sc_embed(token_table, idx_flat, pos_tiled):
    mesh = plsc.VectorSubcoreMesh(core_axis_name="c", subcore_axis_name="s")

    @functools.partial(
        pl.kernel,
        mesh=mesh,
        compiler_params=pltpu.CompilerParams(use_tc_tiling_on_sc=False),
        out_type=jax.ShapeDtypeStruct((TOTAL // 4, 4 * DIM), jnp.float32),
        scratch_types=[
            pltpu.VMEM((CHUNK,), jnp.int32),
            pltpu.VMEM((CHUNK,), jnp.int32),
            pltpu.VMEM((CHUNK, DIM), jnp.float32),
            pltpu.VMEM((CHUNK, DIM), jnp.float32),
            pltpu.VMEM((CHUNK // 4, 4 * DIM), jnp.float32),
            pltpu.VMEM((CHUNK // 4, 4 * DIM), jnp.float32),
            pltpu.SemaphoreType.DMA,
            pltpu.SemaphoreType.DMA,
        ],
    )
    def k(table_hbm, idx_hbm, pos_hbm, out_hbm,
          idx_a, idx_b, rows_a, rows_b, rows_out, pos_v, sem_a, sem_b):
        wid = lax.axis_index("s") * NUM_CORES + lax.axis_index("c")
        base = wid * PER_WORKER
        pltpu.sync_copy(pos_hbm, pos_v)

        def start_gather(ci, idx_v, rows_v, sem):
            off = base + ci * CHUNK
            pltpu.sync_copy(idx_hbm.at[pl.ds(off, CHUNK)], idx_v)
            pltpu.async_copy(table_hbm.at[idx_v], rows_v, sem)

        def wait_gather(idx_v, rows_v, sem):
            pltpu.make_async_copy(table_hbm.at[idx_v], rows_v, sem).wait()

        def add_pos(rows_v):
            @pl.loop(0, CHUNK // 4)
            def _(ro):
                for q in range(4):
                    for c in (0, DIM // 2):
                        dst = (pl.ds(ro, 1), pl.ds(q * DIM + c, 16))
                        src = (pl.ds(4 * ro + q, 1), pl.ds(c, 16))
                        rows_out.at[*dst][...] = (
                            rows_v.at[*src][...] + pos_v.at[*dst][...]
                        )

        def write_out(ci):
            off = base + ci * CHUNK
            pltpu.sync_copy(
                rows_out, out_hbm.at[pl.ds(off // 4, CHUNK // 4)]
            )

        start_gather(0, idx_a, rows_a, sem_a)

        @pl.loop(0, NCHUNK, step=2)
        def _(c0):
            start_gather(c0 + 1, idx_b, rows_b, sem_b)
            wait_gather(idx_a, rows_a, sem_a)
            add_pos(rows_a)

            @pl.when(c0 + 2 < NCHUNK)
            def _():
                start_gather(c0 + 2, idx_a, rows_a, sem_a)

            write_out(c0)
            wait_gather(idx_b, rows_b, sem_b)
            add_pos(rows_b)
            write_out(c0 + 1)

    return k(token_table, idx_flat, pos_tiled)


def kernel(x, token_table, pos_table):
    idx_flat = x.reshape(TOTAL).astype(jnp.int32)
    pos_tiled = jnp.tile(
        pos_table.astype(jnp.float32), (CHUNK // MAXLEN, 1)
    ).reshape(CHUNK // 4, 4 * DIM)
    out = _sc_embed(token_table.astype(jnp.float32), idx_flat, pos_tiled)
    return out.reshape(BATCH, MAXLEN, DIM)
